# baseline (speedup 1.0000x reference)
import functools

import jax
import jax.numpy as jnp
from jax import lax
from jax.experimental import pallas as pl
from jax.experimental.pallas import tpu as pltpu
from jax.experimental.pallas import tpu_sc as plsc

_VOCAB = 1000000
_HID = 64
_B = 1024
_CTX = 10
_NEG = 10

_NC = 2
_NS = 16
_NW = _NC * _NS

_ROWS_PER_W = (_B * _CTX) // _NW
_B_PER_W = _B // _NW


def _sc_gather(ctx_idx, neg_idx, table_in, table_out):
    mesh = plsc.VectorSubcoreMesh(core_axis_name="c", subcore_axis_name="s")

    @functools.partial(
        pl.kernel,
        mesh=mesh,
        compiler_params=pltpu.CompilerParams(use_tc_tiling_on_sc=False),
        out_type=[
            jax.ShapeDtypeStruct((_B, _HID), jnp.float32),
            jax.ShapeDtypeStruct((_B * _NEG, _HID), jnp.float32),
        ],
        scratch_types=[
            pltpu.VMEM((_ROWS_PER_W,), jnp.int32),
            pltpu.VMEM((_ROWS_PER_W, _HID), jnp.float32),
            pltpu.VMEM((_B_PER_W, _HID), jnp.float32),
            pltpu.SemaphoreType.DMA,
        ],
    )
    def k(ctx_idx_hbm, neg_idx_hbm, tin_hbm, tout_hbm,
          ctx_out, neg_out, idx_v, rows_v, acc_v, sem):
        wid = lax.axis_index("s") * _NC + lax.axis_index("c")
        rbase = wid * _ROWS_PER_W
        bbase = wid * _B_PER_W

        pltpu.sync_copy(ctx_idx_hbm.at[pl.ds(rbase, _ROWS_PER_W)], idx_v)
        pltpu.async_copy(tin_hbm.at[idx_v], rows_v, sem).wait()

        def body(b, carry):
            for v in range(_HID // 16):
                sl = pl.ds(v * 16, 16)
                a = rows_v[b * _CTX, sl]
                for j in range(1, _CTX):
                    a = a + rows_v[b * _CTX + j, sl]
                acc_v[b, sl] = a * (1.0 / _CTX)
            return carry

        lax.fori_loop(0, _B_PER_W, body, 0)
        pltpu.sync_copy(acc_v, ctx_out.at[pl.ds(bbase, _B_PER_W)])

        pltpu.sync_copy(neg_idx_hbm.at[pl.ds(rbase, _ROWS_PER_W)], idx_v)
        pltpu.async_copy(tout_hbm.at[idx_v], rows_v, sem).wait()
        pltpu.sync_copy(rows_v, neg_out.at[pl.ds(rbase, _ROWS_PER_W)])

    return k(ctx_idx, neg_idx, table_in, table_out)


_BI = 16


def _tc_body(neg_ref, ctx_ref, out_ref):
    x = lax.dot_general(
        neg_ref[...], ctx_ref[...], (((1,), (1,)), ((), ())),
        preferred_element_type=jnp.float32,
    )
    y = jax.nn.sigmoid(jax.nn.sigmoid(x))
    out_ref[...] = y.reshape(_BI, _NEG, _B).transpose(0, 2, 1)


def _tc_matmul(ctx, neg_rows):
    return pl.pallas_call(
        _tc_body,
        grid=(_B // _BI,),
        in_specs=[
            pl.BlockSpec((_BI * _NEG, _HID), lambda ic: (ic, 0)),
            pl.BlockSpec((_B, _HID), lambda ic: (0, 0)),
        ],
        out_specs=pl.BlockSpec((_BI, _B, _NEG), lambda ic: (ic, 0, 0)),
        out_shape=jax.ShapeDtypeStruct((_B, _B, _NEG), jnp.float32),
    )(neg_rows, ctx)


def kernel(inputs, negative_samples, table_in, table_out):
    ctx_idx = inputs.reshape(-1).astype(jnp.int32)
    neg_idx = negative_samples.reshape(-1).astype(jnp.int32)
    ctx, neg_rows = _sc_gather(ctx_idx, neg_idx, table_in, table_out)
    return _tc_matmul(ctx, neg_rows)


# speedup vs baseline: 1.3199x; 1.3199x over previous
import functools

import jax
import jax.numpy as jnp
from jax import lax
from jax.experimental import pallas as pl
from jax.experimental.pallas import tpu as pltpu
from jax.experimental.pallas import tpu_sc as plsc

_VOCAB = 1000000
_HID = 64
_B = 1024
_CTX = 10
_NEG = 10

_NC = 2
_NS = 16
_NW = _NC * _NS

_ROWS_PER_W = (_B * _CTX) // _NW
_B_PER_W = _B // _NW


_CHUNK = 16


def _sc_gather(ctx_idx, neg_idx, table_in, table_out):
    mesh = plsc.VectorSubcoreMesh(core_axis_name="c", subcore_axis_name="s")

    @functools.partial(
        pl.kernel,
        mesh=mesh,
        out_type=[
            jax.ShapeDtypeStruct((_B, _HID), jnp.float32),
            jax.ShapeDtypeStruct((_B * _NEG, _HID), jnp.float32),
        ],
        scratch_types=[
            pltpu.VMEM((_ROWS_PER_W,), jnp.int32),
            pltpu.VMEM((_ROWS_PER_W, _HID), jnp.float32),
            pltpu.VMEM((_B_PER_W, _HID), jnp.float32),
            pltpu.SemaphoreType.DMA,
        ],
    )
    def k(ctx_idx_hbm, neg_idx_hbm, tin_hbm, tout_hbm,
          ctx_out, neg_out, idx_v, rows_v, acc_v, sem):
        wid = lax.axis_index("s") * _NC + lax.axis_index("c")
        rbase = wid * _ROWS_PER_W
        bbase = wid * _B_PER_W

        def fetch_rows(tbl):
            def round_(c, carry):
                iv = idx_v[pl.ds(c * _CHUNK, _CHUNK)]
                handles = []
                for l in range(_CHUNK):
                    r = iv[l]
                    handles.append(pltpu.async_copy(
                        tbl.at[pl.ds(r, 1)],
                        rows_v.at[pl.ds(c * _CHUNK + l, 1)], sem))
                for h in handles:
                    h.wait()
                return carry
            lax.fori_loop(0, _ROWS_PER_W // _CHUNK, round_, 0)

        pltpu.sync_copy(ctx_idx_hbm.at[pl.ds(rbase, _ROWS_PER_W)], idx_v)
        fetch_rows(tin_hbm)

        def body(b, carry):
            for v in range(_HID // 16):
                sl = pl.ds(v * 16, 16)
                a = rows_v[b * _CTX, sl]
                for j in range(1, _CTX):
                    a = a + rows_v[b * _CTX + j, sl]
                acc_v[b, sl] = a * (1.0 / _CTX)
            return carry

        lax.fori_loop(0, _B_PER_W, body, 0)
        pltpu.sync_copy(acc_v, ctx_out.at[pl.ds(bbase, _B_PER_W)])

        pltpu.sync_copy(neg_idx_hbm.at[pl.ds(rbase, _ROWS_PER_W)], idx_v)
        fetch_rows(tout_hbm)
        pltpu.sync_copy(rows_v, neg_out.at[pl.ds(rbase, _ROWS_PER_W)])

    return k(ctx_idx, neg_idx, table_in, table_out)


_BI = 16


def _tc_body(neg_ref, ctx_ref, out_ref):
    x = lax.dot_general(
        neg_ref[...], ctx_ref[...], (((1,), (1,)), ((), ())),
        preferred_element_type=jnp.float32,
    )
    y = jax.nn.sigmoid(jax.nn.sigmoid(x))
    out_ref[...] = y.reshape(_BI, _NEG, _B).transpose(0, 2, 1)


def _tc_matmul(ctx, neg_rows):
    return pl.pallas_call(
        _tc_body,
        grid=(_B // _BI,),
        in_specs=[
            pl.BlockSpec((_BI * _NEG, _HID), lambda ic: (ic, 0)),
            pl.BlockSpec((_B, _HID), lambda ic: (0, 0)),
        ],
        out_specs=pl.BlockSpec((_BI, _B, _NEG), lambda ic: (ic, 0, 0)),
        out_shape=jax.ShapeDtypeStruct((_B, _B, _NEG), jnp.float32),
    )(neg_rows, ctx)


def kernel(inputs, negative_samples, table_in, table_out):
    ctx_idx = inputs.reshape(-1).astype(jnp.int32)
    neg_idx = negative_samples.reshape(-1).astype(jnp.int32)
    ctx, neg_rows = _sc_gather(ctx_idx, neg_idx, table_in, table_out)
    return _tc_matmul(ctx, neg_rows)
